# Initial kernel scaffold; baseline (speedup 1.0000x reference)
import jax
import jax.numpy as jnp
from jax.experimental import pallas as pl


def kernel(x):
    raise NotImplementedError("write your pallas kernel here")


# speedup vs baseline: 1.7643x; 1.7643x over previous
import functools

import jax
import jax.numpy as jnp
import numpy as np
from jax.experimental import pallas as pl


_N_REST = 126
_N_PAIRS = _N_REST * (_N_REST - 1)
_D_OUT = _N_PAIRS + 2


def _pair_idx_np():
    idx = np.arange(_N_REST)
    i_idx, j_idx = np.meshgrid(idx, idx, indexing="ij")
    mask = i_idx != j_idx
    return i_idx[mask].ravel(), j_idx[mask].ravel()


@functools.lru_cache(maxsize=1)
def _selection_matrix():
    i_idx, j_idx = _pair_idx_np()
    w = np.zeros((128, _D_OUT), np.float32)
    w[0, 0] = 1.0
    w[1, 1] = 1.0
    p = np.arange(_N_PAIRS)
    w[2 + i_idx, 2 + p] = 0.75
    w[2 + j_idx, 2 + p] += 0.25
    return jnp.asarray(w)


def _mm_body(x_ref, w_ref, o_ref):
    o_ref[...] = jnp.dot(
        x_ref[...], w_ref[...], preferred_element_type=jnp.float32
    )


def kernel(x):
    b, d = x.shape
    assert d == 128
    w = _selection_matrix()
    bb = 512
    bc = 2048
    grid = (b // bb, pl.cdiv(_D_OUT, bc))
    return pl.pallas_call(
        _mm_body,
        grid=grid,
        in_specs=[
            pl.BlockSpec((bb, 128), lambda i, j: (i, 0)),
            pl.BlockSpec((128, bc), lambda i, j: (0, j)),
        ],
        out_specs=pl.BlockSpec((bb, bc), lambda i, j: (i, j)),
        out_shape=jax.ShapeDtypeStruct((b, _D_OUT), jnp.float32),
    )(x, w)
